# baseline (speedup 1.0000x reference)
import dataclasses
import functools

import jax
import jax.numpy as jnp
from jax import lax
from jax.experimental import pallas as pl
from jax.experimental.pallas import tpu as pltpu
from jax.experimental.pallas import tpu_sc as plsc

_B = 4
_N = 1024
_H = 12
_NP = _H // 2
_NREP = 4
_NBUCKETS = 32
_TAB = _NBUCKETS * _NBUCKETS
_MAXD = 128
_L = 16
_NC = 2
_NS = 16
_ROWS_PER_W = (_B * _N) // (_NC * _NS)
_RCHUNK = 2
_SHIFT = 127 * 256 + 127


def _rel_bucket_lut():
    rel = jnp.arange(-127, 128, dtype=jnp.int32)
    n = -rel
    nb = _NBUCKETS // 2
    ret = (n < 0).astype(jnp.int32) * nb
    n = jnp.abs(n)
    max_exact = nb // 2
    is_small = n < max_exact
    n_safe = jnp.maximum(n, 1).astype(jnp.float32)
    val_if_large = max_exact + jnp.floor(
        jnp.log(n_safe / max_exact)
        / jnp.log(jnp.float32(_MAXD / max_exact))
        * (nb - max_exact)
    ).astype(jnp.int32)
    val_if_large = jnp.minimum(val_if_large, nb - 1)
    return ret + jnp.where(is_small, n, val_if_large)


def _sc_body(xf_hbm, yf_hbm, lutx_hbm, luty_hbm, rep_hbm, cmp_hbm, out_hbm,
             xf_v, yf_v, c_v, lutx_v, lutyrep_v, tabs, ctabs, rowbufs,
             osem0, osem1):
    cid = lax.axis_index("c")
    sid = lax.axis_index("s")
    wid = sid * _NC + cid
    nslab = _N // _ROWS_PER_W
    b = wid // nslab
    i0 = (wid % nslab) * _ROWS_PER_W

    pltpu.sync_copy(xf_hbm.at[b], xf_v)
    pltpu.sync_copy(yf_hbm.at[b], yf_v)
    pltpu.sync_copy(lutx_hbm, lutx_v)
    pltpu.sync_copy(luty_hbm, lutyrep_v)
    for p in range(_NREP):
        pltpu.sync_copy(rep_hbm.at[p], tabs[p])
    for p in range(_NP - _NREP):
        pltpu.sync_copy(cmp_hbm.at[p], ctabs[p])

    @pl.loop(0, _N, step=_L)
    def _(c):
        s = pl.ds(c, _L)
        xi = (xf_v[s] * float(_MAXD)).astype(jnp.int32)
        yi = (yf_v[s] * float(_MAXD)).astype(jnp.int32)
        c_v[s] = xi * 256 + yi

    osems = (osem0, osem1)
    iota = lax.iota(jnp.int32, _L)
    himask = jnp.int32(-65536)

    @pl.loop(0, _ROWS_PER_W, step=2 * _RCHUNK)
    def _(r4):
        for sub in range(2):
            ibase = i0 + r4 + sub * _RCHUNK
            buf = rowbufs[sub]
            sem = osems[sub]

            @pl.when(r4 >= 2 * _RCHUNK)
            def _():
                for h in range(_H):
                    pltpu.make_async_copy(
                        buf[h],
                        out_hbm.at[b, h, pl.ds(ibase - 2 * _RCHUNK, _RCHUNK)],
                        sem).wait()

            for rr in range(_RCHUNK):
                i = ibase + rr
                iv = jnp.full((_L,), i, dtype=jnp.int32)
                siv = plsc.load_gather(c_v, [iv]) + _SHIFT

                @plsc.parallel_loop(0, _N, step=_L, unroll=4)
                def _(c):
                    s = pl.ds(c, _L)
                    d = siv - c_v[s]
                    dx = jnp.right_shift(d, 8)
                    dyr = jnp.left_shift(jnp.bitwise_and(d, 255), 4) + iota
                    bx512 = plsc.load_gather(lutx_v, [dx])
                    byr = plsc.load_gather(lutyrep_v, [dyr])
                    cidr = bx512 + byr
                    cidx = jnp.right_shift(cidr, 4)
                    for p in range(_NP):
                        if p < _NREP:
                            w = plsc.load_gather(tabs[p], [cidr])
                        else:
                            w = plsc.load_gather(ctabs[p - _NREP], [cidx])
                        buf[2 * p][rr, s] = plsc.bitcast(
                            jnp.bitwise_and(w, himask), jnp.float32)
                        buf[2 * p + 1][rr, s] = plsc.bitcast(
                            jnp.left_shift(w, 16), jnp.float32)

            for h in range(_H):
                pltpu.async_copy(
                    buf[h], out_hbm.at[b, h, pl.ds(ibase, _RCHUNK)], sem)

    for sub in range(2):
        ibase = i0 + _ROWS_PER_W - 2 * _RCHUNK + sub * _RCHUNK
        for h in range(_H):
            pltpu.make_async_copy(
                rowbufs[sub][h],
                out_hbm.at[b, h, pl.ds(ibase, _RCHUNK)],
                osems[sub]).wait()


@jax.jit
def kernel(coords_2d, bias_table):
    lut = _rel_bucket_lut()
    lutx512 = jnp.zeros((256,), jnp.int32).at[:255].set(lut * (_NBUCKETS * _L))
    luty16 = jnp.zeros((256,), jnp.int32).at[:255].set(lut * _L)
    lutyrep = (luty16[:, None] + jnp.arange(_L, dtype=jnp.int32)[None, :]
               ).reshape(256 * _L)

    tb = bias_table.astype(jnp.bfloat16)
    tu = lax.bitcast_convert_type(tb, jnp.uint16).astype(jnp.uint32)
    packed = (tu[:, 0::2] << 16) | tu[:, 1::2]
    packed = packed.T.astype(jnp.int32)
    rep = jnp.broadcast_to(
        packed[:_NREP, :, None], (_NREP, _TAB, _L)).reshape(_NREP, _TAB * _L)
    cmp = packed[_NREP:]

    xf = coords_2d[:, :, 0]
    yf = coords_2d[:, :, 1]

    mesh = plsc.VectorSubcoreMesh(
        core_axis_name="c", subcore_axis_name="s",
        num_cores=_NC, num_subcores=_NS)
    cp = pltpu.CompilerParams()
    if "needs_layout_passes" in pltpu.CompilerParams.__dataclass_fields__:
        cp = dataclasses.replace(cp, needs_layout_passes=False)
    run = pl.kernel(
        _sc_body,
        compiler_params=cp,
        out_type=jax.ShapeDtypeStruct((_B, _H, _N, _N), jnp.float32),
        mesh=mesh,
        scratch_types=[
            pltpu.VMEM((_N,), jnp.float32),
            pltpu.VMEM((_N,), jnp.float32),
            pltpu.VMEM((_N,), jnp.int32),
            pltpu.VMEM((256,), jnp.int32),
            pltpu.VMEM((256 * _L,), jnp.int32),
            [pltpu.VMEM((_TAB * _L,), jnp.int32)
             for _ in range(_NREP)],
            [pltpu.VMEM((_TAB,), jnp.int32)
             for _ in range(_NP - _NREP)],
            [[pltpu.VMEM((_RCHUNK, _N), jnp.float32) for _ in range(_H)]
             for _ in range(2)],
            pltpu.SemaphoreType.DMA,
            pltpu.SemaphoreType.DMA,
        ],
    )
    return run(xf, yf, lutx512, lutyrep, rep, cmp)


# speedup vs baseline: 1.0877x; 1.0877x over previous
import dataclasses
import functools

import jax
import jax.numpy as jnp
from jax import lax
from jax.experimental import pallas as pl
from jax.experimental.pallas import tpu as pltpu
from jax.experimental.pallas import tpu_sc as plsc

_B = 4
_N = 1024
_H = 12
_NP = _H // 2
_NBUCKETS = 32
_TAB = _NBUCKETS * _NBUCKETS
_MAXD = 128
_L = 16
_NC = 2
_NS = 16
_ROWS_PER_W = (_B * _N) // (_NC * _NS)
_SHIFT = 127 * 256 + 127


def _rel_bucket_lut():
    rel = jnp.arange(-127, 128, dtype=jnp.int32)
    n = -rel
    nb = _NBUCKETS // 2
    ret = (n < 0).astype(jnp.int32) * nb
    n = jnp.abs(n)
    max_exact = nb // 2
    is_small = n < max_exact
    n_safe = jnp.maximum(n, 1).astype(jnp.float32)
    val_if_large = max_exact + jnp.floor(
        jnp.log(n_safe / max_exact)
        / jnp.log(jnp.float32(_MAXD / max_exact))
        * (nb - max_exact)
    ).astype(jnp.int32)
    val_if_large = jnp.minimum(val_if_large, nb - 1)
    return ret + jnp.where(is_small, n, val_if_large)


def _sc_body(xf_hbm, yf_hbm, lutrep_hbm, rep_hbm, out_hbm,
             xf_v, yf_v, c_v, lutrep_v, tabs, rowbufs,
             osem0, osem1):
    cid = lax.axis_index("c")
    sid = lax.axis_index("s")
    wid = sid * _NC + cid
    nslab = _N // _ROWS_PER_W
    b = wid // nslab
    i0 = (wid % nslab) * _ROWS_PER_W

    pltpu.sync_copy(xf_hbm.at[b], xf_v)
    pltpu.sync_copy(yf_hbm.at[b], yf_v)
    pltpu.sync_copy(lutrep_hbm, lutrep_v)
    for p in range(_NP):
        pltpu.sync_copy(rep_hbm.at[p], tabs[p])

    @pl.loop(0, _N, step=_L)
    def _(c):
        s = pl.ds(c, _L)
        xi = (xf_v[s] * float(_MAXD)).astype(jnp.int32)
        yi = (yf_v[s] * float(_MAXD)).astype(jnp.int32)
        c_v[s] = xi * 256 + yi

    osems = (osem0, osem1)
    iota = lax.iota(jnp.int32, _L)
    iota32 = iota * 32
    himask = jnp.int32(-65536)

    @pl.loop(0, _ROWS_PER_W, step=2)
    def _(r2):
        for sub in range(2):
            i = i0 + r2 + sub
            buf = rowbufs[sub]
            sem = osems[sub]

            @pl.when(r2 >= 2)
            def _():
                for h in range(_H):
                    pltpu.make_async_copy(
                        buf[h], out_hbm.at[b, h, i - 2], sem).wait()

            iv = jnp.full((_L,), i, dtype=jnp.int32)
            siv = plsc.load_gather(c_v, [iv]) + _SHIFT

            @plsc.parallel_loop(0, _N, step=_L, unroll=4)
            def _(c):
                s = pl.ds(c, _L)
                d = siv - c_v[s]
                dyr = jnp.left_shift(jnp.bitwise_and(d, 255), 4) + iota
                dxr = jnp.bitwise_and(jnp.right_shift(d, 4), 4080) + iota
                gx = plsc.load_gather(lutrep_v, [dxr])
                gy = plsc.load_gather(lutrep_v, [dyr])
                cidr = jnp.left_shift(gx, 5) + gy - iota32
                for p in range(_NP):
                    w = plsc.load_gather(tabs[p], [cidr])
                    buf[2 * p][s] = plsc.bitcast(
                        jnp.bitwise_and(w, himask), jnp.float32)
                    buf[2 * p + 1][s] = plsc.bitcast(
                        jnp.left_shift(w, 16), jnp.float32)

            for h in range(_H):
                pltpu.async_copy(buf[h], out_hbm.at[b, h, i], sem)

    for sub in range(2):
        i = i0 + _ROWS_PER_W - 2 + sub
        for h in range(_H):
            pltpu.make_async_copy(
                rowbufs[sub][h], out_hbm.at[b, h, i], osems[sub]).wait()


@jax.jit
def kernel(coords_2d, bias_table):
    lut = _rel_bucket_lut()
    lut16 = jnp.zeros((256,), jnp.int32).at[:255].set(lut * _L)
    lutrep = (lut16[:, None] + jnp.arange(_L, dtype=jnp.int32)[None, :]
              ).reshape(256 * _L)

    tb = bias_table.astype(jnp.bfloat16)
    tu = lax.bitcast_convert_type(tb, jnp.uint16).astype(jnp.uint32)
    packed = (tu[:, 0::2] << 16) | tu[:, 1::2]
    packed = packed.T.astype(jnp.int32)
    rep = jnp.broadcast_to(
        packed[:, :, None], (_NP, _TAB, _L)).reshape(_NP, _TAB * _L)

    xf = coords_2d[:, :, 0]
    yf = coords_2d[:, :, 1]

    mesh = plsc.VectorSubcoreMesh(
        core_axis_name="c", subcore_axis_name="s",
        num_cores=_NC, num_subcores=_NS)
    cp = pltpu.CompilerParams()
    if "needs_layout_passes" in pltpu.CompilerParams.__dataclass_fields__:
        cp = dataclasses.replace(cp, needs_layout_passes=False)
    run = pl.kernel(
        _sc_body,
        compiler_params=cp,
        out_type=jax.ShapeDtypeStruct((_B, _H, _N, _N), jnp.float32),
        mesh=mesh,
        scratch_types=[
            pltpu.VMEM((_N,), jnp.float32),
            pltpu.VMEM((_N,), jnp.float32),
            pltpu.VMEM((_N,), jnp.int32),
            pltpu.VMEM((256 * _L,), jnp.int32),
            [pltpu.VMEM((_TAB * _L,), jnp.int32)
             for _ in range(_NP)],
            [[pltpu.VMEM((_N,), jnp.float32) for _ in range(_H)]
             for _ in range(2)],
            pltpu.SemaphoreType.DMA,
            pltpu.SemaphoreType.DMA,
        ],
    )
    return run(xf, yf, lutrep, rep)


# speedup vs baseline: 1.0936x; 1.0054x over previous
import dataclasses
import functools

import jax
import jax.numpy as jnp
from jax import lax
from jax.experimental import pallas as pl
from jax.experimental.pallas import tpu as pltpu
from jax.experimental.pallas import tpu_sc as plsc

_B = 4
_N = 1024
_H = 12
_NP = _H // 2
_NBUCKETS = 32
_TAB = _NBUCKETS * _NBUCKETS
_MAXD = 128
_L = 16
_NC = 2
_NS = 16
_ROWS_PER_W = (_B * _N) // (_NC * _NS)
_SHIFT = 127 * 256 + 127


def _rel_bucket_lut():
    rel = jnp.arange(-127, 128, dtype=jnp.int32)
    n = -rel
    nb = _NBUCKETS // 2
    ret = (n < 0).astype(jnp.int32) * nb
    n = jnp.abs(n)
    max_exact = nb // 2
    is_small = n < max_exact
    n_safe = jnp.maximum(n, 1).astype(jnp.float32)
    val_if_large = max_exact + jnp.floor(
        jnp.log(n_safe / max_exact)
        / jnp.log(jnp.float32(_MAXD / max_exact))
        * (nb - max_exact)
    ).astype(jnp.int32)
    val_if_large = jnp.minimum(val_if_large, nb - 1)
    return ret + jnp.where(is_small, n, val_if_large)


def _sc_body(xf_hbm, yf_hbm, lutrep_hbm, rep_hbm, out_hbm,
             xf_v, yf_v, c_v, lutrep_v, tabs, rowbufs,
             osem0, osem1):
    cid = lax.axis_index("c")
    sid = lax.axis_index("s")
    wid = sid * _NC + cid
    nslab = _N // _ROWS_PER_W
    b = wid // nslab
    i0 = (wid % nslab) * _ROWS_PER_W
    rp0 = b * (_H * _N) + i0

    pltpu.sync_copy(xf_hbm.at[b], xf_v)
    pltpu.sync_copy(yf_hbm.at[b], yf_v)
    pltpu.sync_copy(lutrep_hbm, lutrep_v)
    for p in range(_NP):
        pltpu.sync_copy(rep_hbm.at[p], tabs[p])

    @pl.loop(0, _N, step=_L)
    def _(c):
        s = pl.ds(c, _L)
        xi = (xf_v[s] * float(_MAXD)).astype(jnp.int32)
        yi = (yf_v[s] * float(_MAXD)).astype(jnp.int32)
        c_v[s] = xi * 256 + yi

    osems = (osem0, osem1)
    iota = lax.iota(jnp.int32, _L)
    iota32 = iota * 32
    himask = jnp.int32(-65536)

    @pl.loop(0, _ROWS_PER_W, step=2)
    def _(r2):
        for sub in range(2):
            i = i0 + r2 + sub
            rp = rp0 + r2 + sub
            buf = rowbufs[sub]
            sem = osems[sub]

            @pl.when(r2 >= 2)
            def _():
                for h in range(_H):
                    pltpu.make_async_copy(
                        buf[h], out_hbm.at[rp - 2 + h * _N], sem).wait()

            iv = jnp.full((_L,), i, dtype=jnp.int32)
            siv = plsc.load_gather(c_v, [iv]) + _SHIFT

            @plsc.parallel_loop(0, _N, step=_L, unroll=4)
            def _(c):
                s = pl.ds(c, _L)
                d = siv - c_v[s]
                dyr = jnp.left_shift(jnp.bitwise_and(d, 255), 4) + iota
                dxr = jnp.bitwise_and(jnp.right_shift(d, 4), 4080) + iota
                gx = plsc.load_gather(lutrep_v, [dxr])
                gy = plsc.load_gather(lutrep_v, [dyr])
                cidr = jnp.left_shift(gx, 5) + gy - iota32
                for p in range(_NP):
                    w = plsc.load_gather(tabs[p], [cidr])
                    buf[2 * p][s] = plsc.bitcast(
                        jnp.bitwise_and(w, himask), jnp.float32)
                    buf[2 * p + 1][s] = plsc.bitcast(
                        jnp.left_shift(w, 16), jnp.float32)

            for h in range(_H):
                pltpu.async_copy(buf[h], out_hbm.at[rp + h * _N], sem)

    for sub in range(2):
        rp = rp0 + _ROWS_PER_W - 2 + sub
        for h in range(_H):
            pltpu.make_async_copy(
                rowbufs[sub][h], out_hbm.at[rp + h * _N], osems[sub]).wait()


@jax.jit
def kernel(coords_2d, bias_table):
    lut = _rel_bucket_lut()
    lut16 = jnp.zeros((256,), jnp.int32).at[:255].set(lut * _L)
    lutrep = (lut16[:, None] + jnp.arange(_L, dtype=jnp.int32)[None, :]
              ).reshape(256 * _L)

    tb = bias_table.astype(jnp.bfloat16)
    tu = lax.bitcast_convert_type(tb, jnp.uint16).astype(jnp.uint32)
    packed = (tu[:, 0::2] << 16) | tu[:, 1::2]
    packed = packed.T.astype(jnp.int32)
    rep = jnp.broadcast_to(
        packed[:, :, None], (_NP, _TAB, _L)).reshape(_NP, _TAB * _L)

    xf = coords_2d[:, :, 0]
    yf = coords_2d[:, :, 1]

    mesh = plsc.VectorSubcoreMesh(
        core_axis_name="c", subcore_axis_name="s",
        num_cores=_NC, num_subcores=_NS)
    cp = pltpu.CompilerParams()
    if "needs_layout_passes" in pltpu.CompilerParams.__dataclass_fields__:
        cp = dataclasses.replace(cp, needs_layout_passes=False)
    run = pl.kernel(
        _sc_body,
        compiler_params=cp,
        out_type=jax.ShapeDtypeStruct((_B * _H * _N, _N), jnp.float32),
        mesh=mesh,
        scratch_types=[
            pltpu.VMEM((_N,), jnp.float32),
            pltpu.VMEM((_N,), jnp.float32),
            pltpu.VMEM((_N,), jnp.int32),
            pltpu.VMEM((256 * _L,), jnp.int32),
            [pltpu.VMEM((_TAB * _L,), jnp.int32)
             for _ in range(_NP)],
            [[pltpu.VMEM((_N,), jnp.float32) for _ in range(_H)]
             for _ in range(2)],
            pltpu.SemaphoreType.DMA,
            pltpu.SemaphoreType.DMA,
        ],
    )
    out = run(xf, yf, lutrep, rep)
    return out.reshape(_B, _H, _N, _N)


# speedup vs baseline: 1.1223x; 1.0263x over previous
import dataclasses
import functools

import jax
import jax.numpy as jnp
from jax import lax
from jax.experimental import pallas as pl
from jax.experimental.pallas import tpu as pltpu
from jax.experimental.pallas import tpu_sc as plsc

_B = 4
_N = 1024
_H = 12
_NP = _H // 2
_NBUCKETS = 32
_TAB = _NBUCKETS * _NBUCKETS
_MAXD = 128
_L = 16
_NC = 2
_NS = 16
_ROWS_PER_W = (_B * _N) // (_NC * _NS)
_SHIFT = 127 * 256 + 127


def _rel_bucket_lut():
    rel = jnp.arange(-127, 128, dtype=jnp.int32)
    n = -rel
    nb = _NBUCKETS // 2
    ret = (n < 0).astype(jnp.int32) * nb
    n = jnp.abs(n)
    max_exact = nb // 2
    is_small = n < max_exact
    n_safe = jnp.maximum(n, 1).astype(jnp.float32)
    val_if_large = max_exact + jnp.floor(
        jnp.log(n_safe / max_exact)
        / jnp.log(jnp.float32(_MAXD / max_exact))
        * (nb - max_exact)
    ).astype(jnp.int32)
    val_if_large = jnp.minimum(val_if_large, nb - 1)
    return ret + jnp.where(is_small, n, val_if_large)


def _sc_body(xf_hbm, yf_hbm, lutrep_hbm, rep_hbm, out_hbm,
             xf_v, yf_v, c_v, lutrep_v, tabs, rowbufs,
             osem0, osem1):
    cid = lax.axis_index("c")
    sid = lax.axis_index("s")
    wid = sid * _NC + cid
    nslab = _N // _ROWS_PER_W
    b = wid // nslab
    i0 = (wid % nslab) * _ROWS_PER_W
    rp0 = b * (_H * _N) + i0

    pltpu.sync_copy(xf_hbm.at[b], xf_v)
    pltpu.sync_copy(yf_hbm.at[b], yf_v)
    pltpu.sync_copy(lutrep_hbm, lutrep_v)
    for p in range(_NP):
        pltpu.sync_copy(rep_hbm.at[p], tabs[p])

    @pl.loop(0, _N, step=_L)
    def _(c):
        s = pl.ds(c, _L)
        xi = (xf_v[s] * float(_MAXD)).astype(jnp.int32)
        yi = (yf_v[s] * float(_MAXD)).astype(jnp.int32)
        c_v[s] = xi * 256 + yi

    osems = (osem0, osem1)
    iota = lax.iota(jnp.int32, _L)
    iota32 = iota * 32
    himask = jnp.int32(-65536)

    @pl.loop(0, _ROWS_PER_W, step=2)
    def _(r2):
        for sub in range(2):
            i = i0 + r2 + sub
            rp = rp0 + r2 + sub
            buf = rowbufs[sub]
            sem = osems[sub]

            @pl.when(r2 >= 2)
            def _():
                pltpu.make_async_copy(
                    rep_hbm.at[0, pl.ds(0, _H * _N)],
                    tabs[0].at[pl.ds(0, _H * _N)], sem).wait()

            iv = jnp.full((_L,), i, dtype=jnp.int32)
            siv = plsc.load_gather(c_v, [iv]) + _SHIFT

            @plsc.parallel_loop(0, _N, step=_L, unroll=4)
            def _(c):
                s = pl.ds(c, _L)
                d = siv - c_v[s]
                dyr = jnp.left_shift(jnp.bitwise_and(d, 255), 4) + iota
                dxr = jnp.bitwise_and(jnp.right_shift(d, 4), 4080) + iota
                gx = plsc.load_gather(lutrep_v, [dxr])
                gy = plsc.load_gather(lutrep_v, [dyr])
                cidr = jnp.left_shift(gx, 5) + gy - iota32
                for p in range(_NP):
                    w = plsc.load_gather(tabs[p], [cidr])
                    buf[2 * p][s] = plsc.bitcast(
                        jnp.bitwise_and(w, himask), jnp.float32)
                    buf[2 * p + 1][s] = plsc.bitcast(
                        jnp.left_shift(w, 16), jnp.float32)

            for h in range(_H):
                pltpu.async_copy(buf[h], out_hbm.at[rp + h * _N], sem)

    for sub in range(2):
        pltpu.make_async_copy(
            rep_hbm.at[0, pl.ds(0, _H * _N)],
            tabs[0].at[pl.ds(0, _H * _N)], osems[sub]).wait()


@jax.jit
def kernel(coords_2d, bias_table):
    lut = _rel_bucket_lut()
    lut16 = jnp.zeros((256,), jnp.int32).at[:255].set(lut * _L)
    lutrep = (lut16[:, None] + jnp.arange(_L, dtype=jnp.int32)[None, :]
              ).reshape(256 * _L)

    tb = bias_table.astype(jnp.bfloat16)
    tu = lax.bitcast_convert_type(tb, jnp.uint16).astype(jnp.uint32)
    packed = (tu[:, 0::2] << 16) | tu[:, 1::2]
    packed = packed.T.astype(jnp.int32)
    rep = jnp.broadcast_to(
        packed[:, :, None], (_NP, _TAB, _L)).reshape(_NP, _TAB * _L)

    xf = coords_2d[:, :, 0]
    yf = coords_2d[:, :, 1]

    mesh = plsc.VectorSubcoreMesh(
        core_axis_name="c", subcore_axis_name="s",
        num_cores=_NC, num_subcores=_NS)
    cp = pltpu.CompilerParams()
    if "needs_layout_passes" in pltpu.CompilerParams.__dataclass_fields__:
        cp = dataclasses.replace(cp, needs_layout_passes=False)
    run = pl.kernel(
        _sc_body,
        compiler_params=cp,
        out_type=jax.ShapeDtypeStruct((_B * _H * _N, _N), jnp.float32),
        mesh=mesh,
        scratch_types=[
            pltpu.VMEM((_N,), jnp.float32),
            pltpu.VMEM((_N,), jnp.float32),
            pltpu.VMEM((_N,), jnp.int32),
            pltpu.VMEM((256 * _L,), jnp.int32),
            [pltpu.VMEM((_TAB * _L,), jnp.int32)
             for _ in range(_NP)],
            [[pltpu.VMEM((_N,), jnp.float32) for _ in range(_H)]
             for _ in range(2)],
            pltpu.SemaphoreType.DMA,
            pltpu.SemaphoreType.DMA,
        ],
    )
    out = run(xf, yf, lutrep, rep)
    return out.reshape(_B, _H, _N, _N)
